# baseline (speedup 1.0000x reference)
import functools

import jax
import jax.numpy as jnp
from jax import lax
from jax.experimental import pallas as pl
from jax.experimental.pallas import tpu as pltpu
from jax.experimental.pallas import tpu_sc as plsc

_L2 = 1.0
_LINF = 0.02
_K = 2048
_B = 64
_N = 32768
_L = 16
_NW = 32
_ROWS_PER_W = _B // _NW
_NV = _N // _L

_DNUMS = lax.GatherDimensionNumbers(
    offset_dims=(), collapsed_slice_dims=(0,), start_index_map=(0,))


def _butterfly(v, op):
    lane = lax.broadcasted_iota(jnp.int32, (_L,), 0)
    for sh in (8, 4, 2, 1):
        g = lax.gather(v, (lane ^ sh)[:, None], _DNUMS, (1,),
                       mode=lax.GatherScatterMode.PROMISE_IN_BOUNDS)
        v = op(v, g)
    return v


def _sc_body(y_hbm, yh_hbm, mask_hbm, out_hbm, ybuf, yhbuf, mbuf, obuf):
    wid = lax.axis_index("s") * 2 + lax.axis_index("c")

    for r in range(_ROWS_PER_W):
        row = wid * _ROWS_PER_W + r
        pltpu.sync_copy(y_hbm.at[row], ybuf)
        pltpu.sync_copy(yh_hbm.at[row], yhbuf)
        pltpu.sync_copy(mask_hbm.at[row], mbuf)

        def ew(i, carry):
            acc, mx = carry
            s = pl.ds(i * _L, _L)
            c = mbuf[s]
            d = ybuf[s] * c - yhbuf[s] * c
            sq = d * d
            ybuf[s] = sq
            return acc + sq, jnp.maximum(mx, sq)

        sqacc, mx = lax.fori_loop(
            0, _NV, ew,
            (jnp.zeros((_L,), jnp.float32), jnp.zeros((_L,), jnp.float32)),
            unroll=8)
        sqsum_v = _butterfly(sqacc, jnp.add)
        row_max = _butterfly(mx, jnp.maximum) * 1.000001 + 1e-37

        def count_ge(midv):
            def body(i, acc):
                v = ybuf[pl.ds(i * _L, _L)]
                return acc + jnp.where(v >= midv, 1, 0)

            acc = lax.fori_loop(0, _NV, body, jnp.zeros((_L,), jnp.int32),
                                unroll=8)
            return _butterfly(acc, jnp.add)

        def bstep(_, carry):
            lo, hi = carry
            mid = 0.5 * (lo + hi)
            mid = jnp.minimum(jnp.maximum(mid, lo), hi)
            ge = count_ge(mid) >= _K
            lo = jnp.where(ge, mid, lo)
            hi = jnp.where(ge, hi, mid)
            return lo, hi

        lo, hi = lax.fori_loop(
            0, 40, bstep, (jnp.zeros((_L,), jnp.float32), row_max))
        tv = lo

        def fin(i, carry):
            sacc, cacc = carry
            sq = ybuf[pl.ds(i * _L, _L)]
            gt = sq > tv
            sacc = sacc + jnp.where(gt, sq, 0.0)
            cacc = cacc + jnp.where(gt, 1, 0)
            return sacc, cacc

        sacc, cacc = lax.fori_loop(
            0, _NV, fin,
            (jnp.zeros((_L,), jnp.float32), jnp.zeros((_L,), jnp.int32)),
            unroll=8)
        s_gt = _butterfly(sacc, jnp.add)
        c_gt = _butterfly(cacc, jnp.add)
        topk_v = s_gt + (_K - c_gt).astype(jnp.float32) * tv

        lane = lax.broadcasted_iota(jnp.int32, (_L,), 0)
        outv = jnp.where(lane == 0, topk_v,
                         jnp.where(lane == 1, sqsum_v, 0.0))
        obuf[...] = outv
        pltpu.sync_copy(obuf, out_hbm.at[row])


@jax.jit
def kernel(y, yh, mask):
    mesh = plsc.VectorSubcoreMesh(core_axis_name="c", subcore_axis_name="s")
    partials = pl.kernel(
        _sc_body,
        mesh=mesh,
        out_type=jax.ShapeDtypeStruct((_B, _L), jnp.float32),
        scratch_types=[
            pltpu.VMEM((_N,), jnp.float32),
            pltpu.VMEM((_N,), jnp.float32),
            pltpu.VMEM((_N,), jnp.float32),
            pltpu.VMEM((_L,), jnp.float32),
        ],
    )(y, yh, mask)
    l2 = jnp.sum(partials[:, 1]) / (_B * _N)
    linf = jnp.sum(partials[:, 0]) / _B
    return _L2 * l2 + _LINF * linf


# speedup vs baseline: 4.2834x; 4.2834x over previous
import jax
import jax.numpy as jnp
from jax.experimental import pallas as pl

_L2 = 1.0
_LINF = 0.02
_K = 2048


def _body(y_ref, yh_ref, mask_ref, out_ref):
    B, N = y_ref.shape
    m = mask_ref[...]
    d = y_ref[...] * m - yh_ref[...] * m
    sq = d * d
    total = jnp.sum(sq)

    bits = jax.lax.bitcast_convert_type(sq, jnp.int32)

    def count_ge(mid):
        lt = jax.lax.shift_right_logical(bits - mid, 31)
        nsub = 8
        w = N // nsub
        parts = [jnp.sum(lt[:, i * w:(i + 1) * w], axis=1, keepdims=True)
                 for i in range(nsub)]
        while len(parts) > 1:
            parts = [parts[i] + parts[i + 1] for i in range(0, len(parts), 2)]
        return N - parts[0]

    row_max = jnp.max(bits, axis=1, keepdims=True)
    lo = jnp.zeros((B, 1), jnp.int32)
    hi = row_max
    c_lo = jnp.full((B, 1), jnp.int32(N))
    c_hi1 = jnp.ones((B, 1), jnp.float32)

    def _done(lo, hi, c_lo):
        return (c_lo == _K) | (lo >= hi)

    def cond(carry):
        i, lo, hi, c_lo, c_hi1 = carry
        return jnp.any(~_done(lo, hi, c_lo))

    def step(carry):
        i, lo, hi, c_lo, c_hi1 = carry
        frac = jnp.maximum((c_lo - _K).astype(jnp.float32), 0.0) / (
            jnp.maximum(c_lo.astype(jnp.float32) - c_hi1, 1.0))
        m_interp = lo + (frac * (hi + 1 - lo).astype(jnp.float32)).astype(
            jnp.int32)
        m_bisect = lo + ((hi - lo + 1) >> 1)
        mid = jnp.where(i % 3 == 2, m_bisect, m_interp)
        mid = jnp.clip(mid, lo + 1, hi)
        cnt = count_ge(mid)
        upd = ~_done(lo, hi, c_lo)
        ge = cnt >= _K
        lo = jnp.where(upd & ge, mid, lo)
        hi = jnp.where(upd & ~ge, mid - 1, hi)
        c_lo = jnp.where(upd & ge, cnt, c_lo)
        c_hi1 = jnp.where(upd & ~ge, cnt.astype(jnp.float32), c_hi1)
        return i + 1, lo, hi, c_lo, c_hi1

    _, lo, hi, c_lo, c_hi1 = jax.lax.while_loop(
        cond, step, (jnp.int32(0), lo, hi, c_lo, c_hi1))

    sentinel = jnp.int32(0x7FFFFFFF)
    ge_lo = bits >= lo
    t_bits = jnp.min(jnp.where(ge_lo, bits, sentinel), axis=1, keepdims=True)
    t = jax.lax.bitcast_convert_type(t_bits, jnp.float32)

    gt = bits > t_bits
    s_gt = jnp.sum(jnp.where(gt, sq, 0.0), axis=1, keepdims=True)
    c_gt = jnp.sum(gt.astype(jnp.int32), axis=1, keepdims=True)
    topk_sum = s_gt + (_K - c_gt).astype(jnp.float32) * t

    linf = jnp.sum(topk_sum) / B
    l2 = total / (B * N)
    out_ref[...] = jnp.reshape(_L2 * l2 + _LINF * linf, (1, 1))


@jax.jit
def kernel(y, yh, mask):
    res = pl.pallas_call(
        _body,
        out_shape=jax.ShapeDtypeStruct((1, 1), jnp.float32),
    )(y, yh, mask)
    return res[0, 0]
